# Initial kernel scaffold; baseline (speedup 1.0000x reference)
import jax
import jax.numpy as jnp
from jax.experimental import pallas as pl


def kernel(users, adjacent_items, intermediate_items, distant_items, graph_rows, graph_cols, graph_vals, theta_user, theta_item, w1_user, w2_user, w1_item, w2_item):
    raise NotImplementedError("write your pallas kernel here")


# speedup vs baseline: 3.8137x; 3.8137x over previous
import functools

import jax
import jax.numpy as jnp
from jax import lax
from jax.experimental import pallas as pl
from jax.experimental.pallas import tpu as pltpu
from jax.experimental.pallas import tpu_sc as plsc

NUM_USERS = 25000
NUM_ITEMS = 25000
N = NUM_USERS + NUM_ITEMS
C = 64
HALF = 32
N_LAYERS = 2
E = 800000
B = 4096

NSUB = 16
CHUNK = 80
CH_PER_BLK = 25
BLK_EDGES = CH_PER_BLK * CHUNK
NBLK = E // BLK_EDGES
BLK_PER_SUB = NBLK // NSUB
DRN = 3120
ZROWS = 624
TAIL = N - NSUB * DRN

_DNUMS = lax.GatherDimensionNumbers(
    offset_dims=(), collapsed_slice_dims=(0,), start_index_map=(0,))


def _bcast_lane(vec, l):
    return lax.gather(vec, jnp.full((16, 1), l, jnp.int32), _DNUMS, (1,),
                      mode=lax.GatherScatterMode.PROMISE_IN_BOUNDS)


def _spmm_body(mrows, mcols, mvals, table, out,
               dst_b, col_b, val_b, idx_v, rows_v, zbuf, acc, sem):
    c = lax.axis_index("c")
    s = lax.axis_index("s")
    colbase = c * N

    zvec = jnp.zeros((16,), jnp.float32)

    def zb_body(i, carry):
        zbuf[i, pl.ds(0, 16)] = zvec
        zbuf[i, pl.ds(16, 16)] = zvec
        return carry

    lax.fori_loop(0, ZROWS, zb_body, 0)
    for r in range(DRN // ZROWS):
        pltpu.sync_copy(zbuf, acc.at[pl.ds(s * DRN + r * ZROWS, ZROWS)])

    @pl.when(s == 0)
    def _zero_tail():
        pltpu.sync_copy(zbuf.at[pl.ds(0, TAIL)], acc.at[pl.ds(NSUB * DRN, TAIL)])

    plsc.subcore_barrier()

    def blk_body(b, carry):
        blk = s * BLK_PER_SUB + b
        pltpu.sync_copy(mrows.at[blk], dst_b)
        pltpu.sync_copy(mcols.at[blk], col_b)
        pltpu.sync_copy(mvals.at[blk], val_b)

        def ch_body(k, carry2):
            for j in range(CHUNK // 16):
                idx_v[pl.ds(j * 16, 16)] = col_b[k, pl.ds(j * 16, 16)] + colbase
            pltpu.async_copy(table.at[idx_v], rows_v, sem).wait()
            for g in range(CHUNK // 16):
                vv = val_b[k, pl.ds(g * 16, 16)]
                for l in range(16):
                    e = g * 16 + l
                    vb = _bcast_lane(vv, l)
                    rows_v[e, pl.ds(0, 16)] = rows_v[e, pl.ds(0, 16)] * vb
                    rows_v[e, pl.ds(16, 16)] = rows_v[e, pl.ds(16, 16)] * vb
            pltpu.sync_copy(rows_v, acc.at[dst_b.at[k]], add=True)
            return carry2

        lax.fori_loop(0, CH_PER_BLK, ch_body, 0)
        return carry

    lax.fori_loop(0, BLK_PER_SUB, blk_body, 0)
    plsc.subcore_barrier()

    for r in range(DRN // ZROWS):
        off = s * DRN + r * ZROWS
        pltpu.sync_copy(acc.at[pl.ds(off, ZROWS)], zbuf)
        pltpu.sync_copy(zbuf, out.at[pl.ds(c * N + off, ZROWS)])

    @pl.when(s == 0)
    def _drain_tail():
        pltpu.sync_copy(acc.at[pl.ds(NSUB * DRN, TAIL)], zbuf.at[pl.ds(0, TAIL)])
        pltpu.sync_copy(zbuf.at[pl.ds(0, TAIL)], out.at[pl.ds(c * N + NSUB * DRN, TAIL)])


_spmm = pl.kernel(
    _spmm_body,
    out_type=jax.ShapeDtypeStruct((2 * N, HALF), jnp.float32),
    mesh=plsc.VectorSubcoreMesh(core_axis_name="c", subcore_axis_name="s"),
    scratch_types=[
        pltpu.VMEM((CH_PER_BLK, CHUNK), jnp.int32),
        pltpu.VMEM((CH_PER_BLK, CHUNK), jnp.int32),
        pltpu.VMEM((CH_PER_BLK, CHUNK), jnp.float32),
        pltpu.VMEM((CHUNK,), jnp.int32),
        pltpu.VMEM((CHUNK, HALF), jnp.float32),
        pltpu.VMEM((ZROWS, HALF), jnp.float32),
        pltpu.VMEM_SHARED((N, HALF), jnp.float32),
        pltpu.SemaphoreType.DMA,
    ],
    compiler_params=pltpu.CompilerParams(use_tc_tiling_on_sc=False),
    name="spmm_sc",
)


def _multi_hop(vectors, items):
    return jnp.mean(vectors[items], axis=1)


def kernel(users, adjacent_items, intermediate_items, distant_items,
           graph_rows, graph_cols, graph_vals,
           theta_user, theta_item, w1_user, w2_user, w1_item, w2_item):
    tu = jax.nn.softmax(theta_user, axis=-1)
    ti = jax.nn.softmax(theta_item, axis=-1)
    theta0 = jnp.concatenate([tu, ti], axis=0)

    t0s = jnp.concatenate([theta0[:, :HALF], theta0[:, HALF:]], axis=0)
    mrows = graph_rows.astype(jnp.int32).reshape(NBLK, CH_PER_BLK, CHUNK)
    mcols = graph_cols.astype(jnp.int32).reshape(NBLK, CH_PER_BLK, CHUNK)
    mvals = graph_vals.reshape(NBLK, CH_PER_BLK, CHUNK)

    t1s = _spmm(mrows, mcols, mvals, t0s)
    t2s = _spmm(mrows, mcols, mvals, t1s)
    theta1 = jnp.concatenate([t1s[:N], t1s[N:]], axis=1)
    theta2 = jnp.concatenate([t2s[:N], t2s[N:]], axis=1)

    all_theta = [theta0, theta1, theta2]
    all_z = [(tu, ti)]
    for th in (theta1, theta2):
        z_user = jax.nn.sigmoid(th[:NUM_USERS] * w1_user + w2_user)
        z_item = jax.nn.sigmoid(th[NUM_USERS:] * w1_item + w2_item)
        all_z.append((z_user, z_item))

    theta_merge = (all_theta[0] + all_theta[1] + all_theta[2]) / 3.0
    lgc_u = theta_merge[:NUM_USERS]
    lgc_i = theta_merge[NUM_USERS:]
    u = lgc_u[users]
    gf_adj = jax.nn.sigmoid(jnp.sum(u * lgc_i[adjacent_items], axis=-1))
    gf_int = jax.nn.sigmoid(jnp.sum(u * _multi_hop(lgc_i, intermediate_items), axis=-1))
    gf_dis = jax.nn.sigmoid(jnp.sum(u * _multi_hop(lgc_i, distant_items), axis=-1))
    gs_adj, gs_int, gs_dis = [], [], []
    for idx in range(N_LAYERS):
        z1u, z1i = all_z[idx]
        z2u, z2i = all_z[idx + 1]
        gs_adj.append(jnp.sum(z1u[users] * z2i[adjacent_items], axis=-1))
        gs_adj.append(jnp.sum(z2u[users] * z1i[adjacent_items], axis=-1))
        gs_int.append(jnp.sum(z1u[users] * _multi_hop(z2i, intermediate_items), axis=-1))
        gs_int.append(jnp.sum(z2u[users] * _multi_hop(z1i, intermediate_items), axis=-1))
        gs_dis.append(jnp.sum(z1u[users] * _multi_hop(z2i, distant_items), axis=-1))
        gs_dis.append(jnp.sum(z2u[users] * _multi_hop(z1i, distant_items), axis=-1))
    gs_adj = (gs_adj[0] + gs_adj[1] + gs_adj[2] + gs_adj[3]) / 4.0
    gs_int = (gs_int[0] + gs_int[1] + gs_int[2] + gs_int[3]) / 4.0
    gs_dis = (gs_dis[0] + gs_dis[1] + gs_dis[2] + gs_dis[3]) / 4.0
    return jnp.stack([gf_adj, gf_int, gf_dis, gs_adj, gs_int, gs_dis], axis=0)


# speedup vs baseline: 6.4128x; 1.6816x over previous
import functools

import jax
import jax.numpy as jnp
from jax import lax
from jax.experimental import pallas as pl
from jax.experimental.pallas import tpu as pltpu
from jax.experimental.pallas import tpu_sc as plsc

NUM_USERS = 25000
NUM_ITEMS = 25000
N = NUM_USERS + NUM_ITEMS
C = 64
HALF = 32
N_LAYERS = 2
E = 800000
B = 4096

NSUB = 16
NBUF = 5
CHUNK = 80
CH_PER_BLK = 25
BLK_EDGES = CH_PER_BLK * CHUNK
NBLK = E // BLK_EDGES
BLK_PER_SUB = NBLK // NSUB
DRN = 3120
ZROWS = 624
TAIL = N - NSUB * DRN

_DNUMS = lax.GatherDimensionNumbers(
    offset_dims=(), collapsed_slice_dims=(0,), start_index_map=(0,))


def _bcast_lane(vec, l):
    return lax.gather(vec, jnp.full((16, 1), l, jnp.int32), _DNUMS, (1,),
                      mode=lax.GatherScatterMode.PROMISE_IN_BOUNDS)


def _spmm_body(mrows, mcols, mvals, table, out,
               dst_b, col_b, val_b, rows_v, obuf, acc,
               gsem0, gsem1, gsem2, gsem3, gsem4):
    gsem = [gsem0, gsem1, gsem2, gsem3, gsem4]
    c = lax.axis_index("c")
    s = lax.axis_index("s")
    colbase = c * N

    zvec = jnp.zeros((16,), jnp.float32)

    def zb_body(i, carry):
        rows_v[0, i, pl.ds(0, 16)] = zvec
        rows_v[0, i, pl.ds(16, 16)] = zvec
        return carry

    lax.fori_loop(0, CHUNK, zb_body, 0)
    for r in range(DRN // CHUNK):
        pltpu.sync_copy(rows_v.at[0], acc.at[pl.ds(s * DRN + r * CHUNK, CHUNK)])

    @pl.when(s == 0)
    def _zero_tail():
        pltpu.sync_copy(rows_v.at[0], acc.at[pl.ds(NSUB * DRN, TAIL)])

    plsc.subcore_barrier()

    def fire(k, slot):
        return pltpu.async_copy(table.at[col_b.at[k]], rows_v.at[slot],
                                gsem[slot])

    def blk_body(b, carry):
        blk = s * BLK_PER_SUB + b
        pltpu.sync_copy(mrows.at[blk], dst_b)
        pltpu.sync_copy(mcols.at[blk], col_b)
        pltpu.sync_copy(mvals.at[blk], val_b)

        def off_body(k, carry2):
            for j in range(CHUNK // 16):
                col_b[k, pl.ds(j * 16, 16)] = col_b[k, pl.ds(j * 16, 16)] + colbase
            return carry2

        lax.fori_loop(0, CH_PER_BLK, off_body, 0)
        for slot in range(NBUF):
            fire(slot, slot)

        def grp_body(gg, carry2):
            for slot in range(NBUF):
                k = gg * NBUF + slot
                pltpu.make_async_copy(table.at[col_b.at[k]], rows_v.at[slot],
                                      gsem[slot]).wait()
                for g in range(CHUNK // 16):
                    vv = val_b[k, pl.ds(g * 16, 16)]
                    for l in range(16):
                        e = g * 16 + l
                        vb = _bcast_lane(vv, l)
                        obuf[e, pl.ds(0, 16)] = rows_v[slot, e, pl.ds(0, 16)] * vb
                        obuf[e, pl.ds(16, 16)] = rows_v[slot, e, pl.ds(16, 16)] * vb
                nk = k + NBUF

                @pl.when(nk < CH_PER_BLK)
                def _refire():
                    fire(nk, slot)

                pltpu.sync_copy(obuf, acc.at[dst_b.at[k]], add=True)
            return carry2

        lax.fori_loop(0, CH_PER_BLK // NBUF, grp_body, 0)
        return carry

    lax.fori_loop(0, BLK_PER_SUB, blk_body, 0)
    plsc.subcore_barrier()

    for r in range(DRN // CHUNK):
        off = s * DRN + r * CHUNK
        pltpu.sync_copy(acc.at[pl.ds(off, CHUNK)], rows_v.at[0])
        pltpu.sync_copy(rows_v.at[0], out.at[pl.ds(c * N + off, CHUNK)])

    @pl.when(s == 0)
    def _drain_tail():
        pltpu.sync_copy(acc.at[pl.ds(NSUB * DRN, TAIL)], rows_v.at[0])
        pltpu.sync_copy(rows_v.at[0], out.at[pl.ds(c * N + NSUB * DRN, TAIL)])


_spmm = pl.kernel(
    _spmm_body,
    out_type=jax.ShapeDtypeStruct((2 * N, HALF), jnp.float32),
    mesh=plsc.VectorSubcoreMesh(core_axis_name="c", subcore_axis_name="s"),
    scratch_types=[
        pltpu.VMEM((CH_PER_BLK, CHUNK), jnp.int32),
        pltpu.VMEM((CH_PER_BLK, CHUNK), jnp.int32),
        pltpu.VMEM((CH_PER_BLK, CHUNK), jnp.float32),
        pltpu.VMEM((NBUF, CHUNK, HALF), jnp.float32),
        pltpu.VMEM((CHUNK, HALF), jnp.float32),
        pltpu.VMEM_SHARED((N, HALF), jnp.float32),
    ] + [pltpu.SemaphoreType.DMA] * NBUF,
    compiler_params=pltpu.CompilerParams(use_tc_tiling_on_sc=False),
    name="spmm_sc",
)


def _multi_hop(vectors, items):
    return jnp.mean(vectors[items], axis=1)


def kernel(users, adjacent_items, intermediate_items, distant_items,
           graph_rows, graph_cols, graph_vals,
           theta_user, theta_item, w1_user, w2_user, w1_item, w2_item):
    tu = jax.nn.softmax(theta_user, axis=-1)
    ti = jax.nn.softmax(theta_item, axis=-1)
    theta0 = jnp.concatenate([tu, ti], axis=0)

    t0s = jnp.concatenate([theta0[:, :HALF], theta0[:, HALF:]], axis=0)
    mrows = graph_rows.astype(jnp.int32).reshape(NBLK, CH_PER_BLK, CHUNK)
    mcols = graph_cols.astype(jnp.int32).reshape(NBLK, CH_PER_BLK, CHUNK)
    mvals = graph_vals.reshape(NBLK, CH_PER_BLK, CHUNK)

    t1s = _spmm(mrows, mcols, mvals, t0s)
    t2s = _spmm(mrows, mcols, mvals, t1s)
    theta1 = jnp.concatenate([t1s[:N], t1s[N:]], axis=1)
    theta2 = jnp.concatenate([t2s[:N], t2s[N:]], axis=1)

    all_theta = [theta0, theta1, theta2]
    all_z = [(tu, ti)]
    for th in (theta1, theta2):
        z_user = jax.nn.sigmoid(th[:NUM_USERS] * w1_user + w2_user)
        z_item = jax.nn.sigmoid(th[NUM_USERS:] * w1_item + w2_item)
        all_z.append((z_user, z_item))

    theta_merge = (all_theta[0] + all_theta[1] + all_theta[2]) / 3.0
    lgc_u = theta_merge[:NUM_USERS]
    lgc_i = theta_merge[NUM_USERS:]
    u = lgc_u[users]
    gf_adj = jax.nn.sigmoid(jnp.sum(u * lgc_i[adjacent_items], axis=-1))
    gf_int = jax.nn.sigmoid(jnp.sum(u * _multi_hop(lgc_i, intermediate_items), axis=-1))
    gf_dis = jax.nn.sigmoid(jnp.sum(u * _multi_hop(lgc_i, distant_items), axis=-1))
    gs_adj, gs_int, gs_dis = [], [], []
    for idx in range(N_LAYERS):
        z1u, z1i = all_z[idx]
        z2u, z2i = all_z[idx + 1]
        gs_adj.append(jnp.sum(z1u[users] * z2i[adjacent_items], axis=-1))
        gs_adj.append(jnp.sum(z2u[users] * z1i[adjacent_items], axis=-1))
        gs_int.append(jnp.sum(z1u[users] * _multi_hop(z2i, intermediate_items), axis=-1))
        gs_int.append(jnp.sum(z2u[users] * _multi_hop(z1i, intermediate_items), axis=-1))
        gs_dis.append(jnp.sum(z1u[users] * _multi_hop(z2i, distant_items), axis=-1))
        gs_dis.append(jnp.sum(z2u[users] * _multi_hop(z1i, distant_items), axis=-1))
    gs_adj = (gs_adj[0] + gs_adj[1] + gs_adj[2] + gs_adj[3]) / 4.0
    gs_int = (gs_int[0] + gs_int[1] + gs_int[2] + gs_int[3]) / 4.0
    gs_dis = (gs_dis[0] + gs_dis[1] + gs_dis[2] + gs_dis[3]) / 4.0
    return jnp.stack([gf_adj, gf_int, gf_dis, gs_adj, gs_int, gs_dis], axis=0)


# speedup vs baseline: 9.3711x; 1.4613x over previous
import functools

import jax
import jax.numpy as jnp
from jax import lax
from jax.experimental import pallas as pl
from jax.experimental.pallas import tpu as pltpu
from jax.experimental.pallas import tpu_sc as plsc

NUM_USERS = 25000
NUM_ITEMS = 25000
N = NUM_USERS + NUM_ITEMS
C = 64
HALF = 32
N_LAYERS = 2
E = 800000
B = 4096

NSUB = 16
NBUF = 5
CHUNK = 80
CH_PER_BLK = 25
BLK_EDGES = CH_PER_BLK * CHUNK
NBLK = E // BLK_EDGES
BLK_PER_SUB = NBLK // NSUB
DRN = 3120
ZROWS = 624
TAIL = N - NSUB * DRN

_DNUMS = lax.GatherDimensionNumbers(
    offset_dims=(), collapsed_slice_dims=(0,), start_index_map=(0,))


def _bcast_lane(vec, l):
    return lax.gather(vec, jnp.full((16, 1), l, jnp.int32), _DNUMS, (1,),
                      mode=lax.GatherScatterMode.PROMISE_IN_BOUNDS)


def _spmm_body(mrows, mcols, mvals, table, out,
               dst_b, col_b, val_b, rows_v, obuf, acc,
               gsem0, gsem1, gsem2, gsem3, gsem4):
    gsem = [gsem0, gsem1, gsem2, gsem3, gsem4]
    c = lax.axis_index("c")
    s = lax.axis_index("s")
    colbase = c * N

    zvec = jnp.zeros((16,), jnp.float32)

    def zb_body(i, carry):
        rows_v[0, i, pl.ds(0, 16)] = zvec
        rows_v[0, i, pl.ds(16, 16)] = zvec
        return carry

    lax.fori_loop(0, CHUNK, zb_body, 0)
    for r in range(DRN // CHUNK):
        pltpu.sync_copy(rows_v.at[0], acc.at[pl.ds(s * DRN + r * CHUNK, CHUNK)])

    @pl.when(s == 0)
    def _zero_tail():
        pltpu.sync_copy(rows_v.at[0], acc.at[pl.ds(NSUB * DRN, TAIL)])

    plsc.subcore_barrier()

    def fire(k, slot):
        return pltpu.async_copy(table.at[col_b.at[k]], rows_v.at[slot],
                                gsem[slot])

    def blk_body(b, carry):
        blk = s * BLK_PER_SUB + b
        pltpu.sync_copy(mrows.at[blk], dst_b)
        pltpu.sync_copy(mcols.at[blk], col_b)
        pltpu.sync_copy(mvals.at[blk], val_b)

        def off_body(k, carry2):
            for j in range(CHUNK // 16):
                col_b[k, pl.ds(j * 16, 16)] = col_b[k, pl.ds(j * 16, 16)] + colbase
            return carry2

        lax.fori_loop(0, CH_PER_BLK, off_body, 0)
        for slot in range(NBUF):
            fire(slot, slot)

        def grp_body(gg, carry2):
            for slot in range(NBUF):
                k = gg * NBUF + slot
                pltpu.make_async_copy(table.at[col_b.at[k]], rows_v.at[slot],
                                      gsem[slot]).wait()
                for g in range(CHUNK // 16):
                    vv = val_b[k, pl.ds(g * 16, 16)]
                    for l in range(16):
                        e = g * 16 + l
                        vb = _bcast_lane(vv, l)
                        obuf[e, pl.ds(0, 16)] = rows_v[slot, e, pl.ds(0, 16)] * vb
                        obuf[e, pl.ds(16, 16)] = rows_v[slot, e, pl.ds(16, 16)] * vb
                nk = k + NBUF

                @pl.when(nk < CH_PER_BLK)
                def _refire():
                    fire(nk, slot)

                pltpu.sync_copy(obuf, acc.at[dst_b.at[k]], add=True)
            return carry2

        lax.fori_loop(0, CH_PER_BLK // NBUF, grp_body, 0)
        return carry

    lax.fori_loop(0, BLK_PER_SUB, blk_body, 0)
    plsc.subcore_barrier()

    for r in range(DRN // CHUNK):
        off = s * DRN + r * CHUNK
        pltpu.sync_copy(acc.at[pl.ds(off, CHUNK)], rows_v.at[0])
        pltpu.sync_copy(rows_v.at[0], out.at[pl.ds(c * N + off, CHUNK)])

    @pl.when(s == 0)
    def _drain_tail():
        pltpu.sync_copy(acc.at[pl.ds(NSUB * DRN, TAIL)], rows_v.at[0])
        pltpu.sync_copy(rows_v.at[0], out.at[pl.ds(c * N + NSUB * DRN, TAIL)])


_spmm = pl.kernel(
    _spmm_body,
    out_type=jax.ShapeDtypeStruct((2 * N, HALF), jnp.float32),
    mesh=plsc.VectorSubcoreMesh(core_axis_name="c", subcore_axis_name="s"),
    scratch_types=[
        pltpu.VMEM((CH_PER_BLK, CHUNK), jnp.int32),
        pltpu.VMEM((CH_PER_BLK, CHUNK), jnp.int32),
        pltpu.VMEM((CH_PER_BLK, CHUNK), jnp.float32),
        pltpu.VMEM((NBUF, CHUNK, HALF), jnp.float32),
        pltpu.VMEM((CHUNK, HALF), jnp.float32),
        pltpu.VMEM_SHARED((N, HALF), jnp.float32),
    ] + [pltpu.SemaphoreType.DMA] * NBUF,
    compiler_params=pltpu.CompilerParams(use_tc_tiling_on_sc=False),
    name="spmm_sc",
)


SRB = 2000


def _softmax_body(x_ref, o_ref):
    h = pl.program_id(0)
    x = x_ref[...]
    m = jnp.max(x, axis=-1, keepdims=True)
    e = jnp.exp(x - m)
    sm = e / jnp.sum(e, axis=-1, keepdims=True)
    o_ref[...] = jnp.where(h == 0, sm[:, :HALF], sm[:, HALF:])


_softmax = pl.pallas_call(
    _softmax_body,
    grid=(2, N // SRB),
    in_specs=[pl.BlockSpec((SRB, C), lambda h, i: (i, 0))],
    out_specs=pl.BlockSpec((SRB, HALF), lambda h, i: (h * (N // SRB) + i, 0)),
    out_shape=jax.ShapeDtypeStruct((2 * N, HALF), jnp.float32),
)


NSLOT = 7
ESUB = B // NSUB
ECH = 64
NCH = ESUB // ECH
_SLOT_ACC = [None, (0, 1.0), (1, 0.5), (1, 0.5),
             (2, 1.0 / 3.0), (2, 1.0 / 3.0), (2, 1.0 / 3.0)]
_SLOT_GS = [None, (3, 0.25), (4, 0.125), (4, 0.125),
            (5, 1.0 / 12.0), (5, 1.0 / 12.0), (5, 1.0 / 12.0)]


def _sig(v):
    return 1.0 / (1.0 + jnp.exp(-v))


def _batch_body(idx_raw, t0s, t1s, t2s, wu, wi, P,
                idx_b, idx_sb, tb, wb, rb, dsem):
    c = lax.axis_index("c")
    s = lax.axis_index("s")
    pltpu.sync_copy(idx_raw.at[:, pl.ds(s * ESUB, ESUB)], idx_b)

    def chunk(ch, carry):
        for t in range(NSLOT):
            base = c * N + (NUM_USERS if t else 0)
            for j in range(ECH // 16):
                idx_sb[t, pl.ds(j * 16, 16)] = (
                    idx_b[t, pl.ds(ch * ECH + j * 16, 16)] + base)
        fired = []
        for t in range(NSLOT):
            for l, tab in enumerate((t0s, t1s, t2s)):
                fired.append(pltpu.async_copy(tab.at[idx_sb.at[t]],
                                              tb.at[l, t], dsem))
            wtab = wu if t == 0 else wi
            fired.append(pltpu.async_copy(
                wtab.at[idx_b.at[t, pl.ds(ch * ECH, ECH)]], wb.at[t], dsem))
        for d in fired:
            d.wait()

        def elem(e, carry2):
            wv = wb[0, e, pl.ds(0, 16)]
            w1 = _bcast_lane(wv, 0)
            w2 = _bcast_lane(wv, 8)
            u0 = [tb[0, 0, e, pl.ds(0, 16)], tb[0, 0, e, pl.ds(16, 16)]]
            u1 = [tb[1, 0, e, pl.ds(0, 16)], tb[1, 0, e, pl.ds(16, 16)]]
            u2 = [tb[2, 0, e, pl.ds(0, 16)], tb[2, 0, e, pl.ds(16, 16)]]
            third = 1.0 / 3.0
            mu = [(u0[0] + u1[0] + u2[0]) * third,
                  (u0[1] + u1[1] + u2[1]) * third]
            z0u = u0
            z1u = [_sig(u1[0] * w1 + w2), _sig(u1[1] * w1 + w2)]
            z2u = [_sig(u2[0] * w1 + w2), _sig(u2[1] * w1 + w2)]
            accs = [jnp.zeros((16,), jnp.float32) for _ in range(6)]
            for t in range(1, NSLOT):
                wv_t = wb[t, e, pl.ds(0, 16)]
                w1t = _bcast_lane(wv_t, 0)
                w2t = _bcast_lane(wv_t, 8)
                v0 = [tb[0, t, e, pl.ds(0, 16)], tb[0, t, e, pl.ds(16, 16)]]
                v1 = [tb[1, t, e, pl.ds(0, 16)], tb[1, t, e, pl.ds(16, 16)]]
                v2 = [tb[2, t, e, pl.ds(0, 16)], tb[2, t, e, pl.ds(16, 16)]]
                mv = [(v0[0] + v1[0] + v2[0]) * third,
                      (v0[1] + v1[1] + v2[1]) * third]
                z1v = [_sig(v1[0] * w1t + w2t), _sig(v1[1] * w1t + w2t)]
                z2v = [_sig(v2[0] * w1t + w2t), _sig(v2[1] * w1t + w2t)]
                a = mu[0] * mv[0] + mu[1] * mv[1]
                bsum = (z0u[0] * z1v[0] + z0u[1] * z1v[1]
                        + z1u[0] * v0[0] + z1u[1] * v0[1]
                        + z1u[0] * z2v[0] + z1u[1] * z2v[1]
                        + z2u[0] * z1v[0] + z2u[1] * z1v[1])
                j, sc = _SLOT_ACC[t]
                accs[j] = accs[j] + a * sc
                j, sc = _SLOT_GS[t]
                accs[j] = accs[j] + bsum * sc
            for j in range(6):
                rb[j, e, pl.ds(0, 16)] = accs[j]
            return carry2

        lax.fori_loop(0, ECH, elem, 0)
        pltpu.sync_copy(rb, P.at[c, :, pl.ds(s * ESUB + ch * ECH, ECH)])
        return carry

    lax.fori_loop(0, NCH, chunk, 0)


_batch = pl.kernel(
    _batch_body,
    out_type=jax.ShapeDtypeStruct((2, 6, B, 16), jnp.float32),
    mesh=plsc.VectorSubcoreMesh(core_axis_name="c", subcore_axis_name="s"),
    scratch_types=[
        pltpu.VMEM((NSLOT, ESUB), jnp.int32),
        pltpu.VMEM((NSLOT, ECH), jnp.int32),
        pltpu.VMEM((3, NSLOT, ECH, HALF), jnp.float32),
        pltpu.VMEM((NSLOT, ECH, 16), jnp.float32),
        pltpu.VMEM((6, ECH, 16), jnp.float32),
        pltpu.SemaphoreType.DMA,
    ],
    compiler_params=pltpu.CompilerParams(use_tc_tiling_on_sc=False),
    name="batch_sc",
)


def _combine_body(p_ref, o_ref):
    p = p_ref[...]
    S = jnp.sum(p, axis=(0, 3))
    o_ref[...] = jnp.concatenate(
        [jax.nn.sigmoid(S[:3]), S[3:]], axis=0)


_combine = pl.pallas_call(
    _combine_body,
    out_shape=jax.ShapeDtypeStruct((6, B), jnp.float32),
)


def kernel(users, adjacent_items, intermediate_items, distant_items,
           graph_rows, graph_cols, graph_vals,
           theta_user, theta_item, w1_user, w2_user, w1_item, w2_item):
    theta_cat = jnp.concatenate([theta_user, theta_item], axis=0)
    t0s = _softmax(theta_cat)

    mrows = graph_rows.astype(jnp.int32).reshape(NBLK, CH_PER_BLK, CHUNK)
    mcols = graph_cols.astype(jnp.int32).reshape(NBLK, CH_PER_BLK, CHUNK)
    mvals = graph_vals.reshape(NBLK, CH_PER_BLK, CHUNK)
    t1s = _spmm(mrows, mcols, mvals, t0s)
    t2s = _spmm(mrows, mcols, mvals, t1s)

    idx_raw = jnp.stack([
        users, adjacent_items,
        intermediate_items[:, 0], intermediate_items[:, 1],
        distant_items[:, 0], distant_items[:, 1], distant_items[:, 2],
    ], axis=0).astype(jnp.int32)
    wu = jnp.concatenate([jnp.broadcast_to(w1_user, (NUM_USERS, 8)),
                          jnp.broadcast_to(w2_user, (NUM_USERS, 8))], axis=1)
    wi = jnp.concatenate([jnp.broadcast_to(w1_item, (NUM_ITEMS, 8)),
                          jnp.broadcast_to(w2_item, (NUM_ITEMS, 8))], axis=1)
    partials = _batch(idx_raw, t0s, t1s, t2s, wu, wi)
    return _combine(partials)
